# baseline (speedup 1.0000x reference)
import functools

import jax
import jax.numpy as jnp
from jax.experimental import pallas as pl
from jax.experimental.pallas import tpu as pltpu

_TK = 2000
_BIG = 1e30


def _nn_kernel(d_ref, l_ref, qt2_ref, out_ref, acc0_ref, acc1_ref, *,
               n_tiles):
    k = pl.program_id(0)

    @pl.when(k == 0)
    def _init():
        acc0_ref[...] = jnp.full(acc0_ref.shape, jnp.inf, jnp.float32)
        acc1_ref[...] = jnp.full(acc1_ref.shape, jnp.inf, jnp.float32)

    d = d_ref[...]
    lab = l_ref[...]
    d2 = jnp.sum(d * d, axis=1, keepdims=True)
    p2 = jax.lax.dot_general(d, qt2_ref[...], (((1,), (0,)), ((), ())),
                             preferred_element_type=jnp.float32)

    b0 = d2 + lab * _BIG
    b1 = d2 + (1.0 - lab) * _BIG
    m0 = jnp.min(b0 - p2, axis=0, keepdims=True)
    m1 = jnp.min(b1 - p2, axis=0, keepdims=True)
    acc0_ref[...] = jnp.minimum(acc0_ref[...], m0)
    acc1_ref[...] = jnp.minimum(acc1_ref[...], m1)

    @pl.when(k == n_tiles - 1)
    def _emit():
        out_ref[...] = jnp.where(acc1_ref[...] < acc0_ref[...], 1.0, 0.0)


def kernel(input, data, labels):
    q, ddim = input.shape
    n = data.shape[0]
    n_tiles = pl.cdiv(n, _TK)
    qt2 = (2.0 * input).T
    lab2d = labels.reshape(n, 1)

    pred = pl.pallas_call(
        functools.partial(_nn_kernel, n_tiles=n_tiles),
        grid=(n_tiles,),
        in_specs=[
            pl.BlockSpec((_TK, ddim), lambda k: (k, 0)),
            pl.BlockSpec((_TK, 1), lambda k: (k, 0)),
            pl.BlockSpec((ddim, q), lambda k: (0, 0)),
        ],
        out_specs=pl.BlockSpec((1, q), lambda k: (0, 0)),
        out_shape=jax.ShapeDtypeStruct((1, q), jnp.float32),
        scratch_shapes=[pltpu.VMEM((1, q), jnp.float32),
                        pltpu.VMEM((1, q), jnp.float32)],
        compiler_params=pltpu.CompilerParams(
            dimension_semantics=("arbitrary",)),
    )(data, lab2d, qt2)

    return (pred.reshape(q, 1), jnp.asarray(0.0, jnp.float32))


# speedup vs baseline: 1.2677x; 1.2677x over previous
import functools

import jax
import jax.numpy as jnp
from jax.experimental import pallas as pl
from jax.experimental.pallas import tpu as pltpu

_TK = 2000
_BIG = 1e30


def _nn_kernel(d_ref, l_ref, qt2_ref, out_ref, acc0_ref, acc1_ref, *,
               n_tiles):
    k = pl.program_id(0)

    @pl.when(k == 0)
    def _init():
        acc0_ref[...] = jnp.full(acc0_ref.shape, jnp.inf, jnp.float32)
        acc1_ref[...] = jnp.full(acc1_ref.shape, jnp.inf, jnp.float32)

    d = d_ref[...]
    lab = jnp.transpose(l_ref[0])
    d2 = jnp.sum(d * d, axis=1, keepdims=True)
    p2 = jax.lax.dot_general(d, qt2_ref[...], (((1,), (0,)), ((), ())),
                             preferred_element_type=jnp.float32)

    b0 = d2 + lab * _BIG
    b1 = d2 + (1.0 - lab) * _BIG
    m0 = jnp.min(b0 - p2, axis=0, keepdims=True)
    m1 = jnp.min(b1 - p2, axis=0, keepdims=True)
    acc0_ref[...] = jnp.minimum(acc0_ref[...], m0)
    acc1_ref[...] = jnp.minimum(acc1_ref[...], m1)

    @pl.when(k == n_tiles - 1)
    def _emit():
        out_ref[...] = jnp.where(acc1_ref[...] < acc0_ref[...], 1.0, 0.0)


def kernel(input, data, labels):
    q, ddim = input.shape
    n = data.shape[0]
    n_tiles = pl.cdiv(n, _TK)
    qt2 = (2.0 * input).T
    lab3d = labels.reshape(n_tiles, 1, _TK)

    pred = pl.pallas_call(
        functools.partial(_nn_kernel, n_tiles=n_tiles),
        grid=(n_tiles,),
        in_specs=[
            pl.BlockSpec((_TK, ddim), lambda k: (k, 0)),
            pl.BlockSpec((1, 1, _TK), lambda k: (k, 0, 0)),
            pl.BlockSpec((ddim, q), lambda k: (0, 0)),
        ],
        out_specs=pl.BlockSpec((1, q), lambda k: (0, 0)),
        out_shape=jax.ShapeDtypeStruct((1, q), jnp.float32),
        scratch_shapes=[pltpu.VMEM((1, q), jnp.float32),
                        pltpu.VMEM((1, q), jnp.float32)],
        compiler_params=pltpu.CompilerParams(
            dimension_semantics=("arbitrary",)),
    )(data, lab3d, qt2)

    return (pred.reshape(q, 1), jnp.asarray(0.0, jnp.float32))


# speedup vs baseline: 1.5655x; 1.2350x over previous
import functools

import jax
import jax.numpy as jnp
from jax.experimental import pallas as pl
from jax.experimental.pallas import tpu as pltpu

_TK = 5000
_C = 2048.0
_INF = float("inf")


def _nn_kernel(d_ref, l_ref, rhs_ref, out_ref, accmin_ref, accmax_ref, *,
               n_tiles):
    k = pl.program_id(0)

    @pl.when(k == 0)
    def _init():
        accmin_ref[...] = jnp.full(accmin_ref.shape, _INF, jnp.float32)
        accmax_ref[...] = jnp.full(accmax_ref.shape, -_INF, jnp.float32)

    d = d_ref[...]
    lab = jnp.transpose(l_ref[0])
    sig = 1.0 - (lab + lab)
    sd = sig * d
    sq = sd * d - sig * (_C / 128.0)
    lhs = jnp.concatenate([sd, sq], axis=1)
    key = jax.lax.dot_general(lhs, rhs_ref[...], (((1,), (0,)), ((), ())),
                              preferred_element_type=jnp.float32)
    accmin_ref[...] = jnp.minimum(accmin_ref[...],
                                  jnp.min(key, axis=0, keepdims=True))
    accmax_ref[...] = jnp.maximum(accmax_ref[...],
                                  jnp.max(key, axis=0, keepdims=True))

    @pl.when(k == n_tiles - 1)
    def _emit():
        out_ref[...] = jnp.where(accmax_ref[...] > -accmin_ref[...], 1.0, 0.0)


def kernel(input, data, labels):
    q, ddim = input.shape
    n = data.shape[0]
    n_tiles = pl.cdiv(n, _TK)
    rhs = jnp.concatenate([(-2.0) * input.T,
                           jnp.ones((ddim, q), jnp.float32)], axis=0)
    lab3d = labels.reshape(n_tiles, 1, _TK)

    pred = pl.pallas_call(
        functools.partial(_nn_kernel, n_tiles=n_tiles),
        grid=(n_tiles,),
        in_specs=[
            pl.BlockSpec((_TK, ddim), lambda k: (k, 0)),
            pl.BlockSpec((1, 1, _TK), lambda k: (k, 0, 0)),
            pl.BlockSpec((2 * ddim, q), lambda k: (0, 0)),
        ],
        out_specs=pl.BlockSpec((1, q), lambda k: (0, 0)),
        out_shape=jax.ShapeDtypeStruct((1, q), jnp.float32),
        scratch_shapes=[pltpu.VMEM((1, q), jnp.float32),
                        pltpu.VMEM((1, q), jnp.float32)],
        compiler_params=pltpu.CompilerParams(
            dimension_semantics=("arbitrary",)),
    )(data, lab3d, rhs)

    return (pred.reshape(q, 1), jnp.asarray(0.0, jnp.float32))
